# baseline (speedup 1.0000x reference)
import jax
import jax.numpy as jnp
from jax import lax
from jax.experimental import pallas as pl
from jax.experimental.pallas import tpu as pltpu
from jax.experimental.pallas import tpu_sc as plsc

_VOCAB = 8192
_D = 768
_ALPHA = 0.1
_MASK_TOK = 2
_TILE = 2048

_NC, _NS, _L = 1, 16, 16
_NW = _NC * _NS
_BL = 2 * 2048
_XPW = _BL // _NW

_mesh = plsc.VectorSubcoreMesh(core_axis_name="c", subcore_axis_name="s", num_cores=1)


def _matvec_body(emb_ref, W_ref, b_ref, p_ref, bestv_ref, besti_ref):
    j = pl.program_id(0)
    nj = pl.num_programs(0)

    v = emb_ref[_MASK_TOK : _MASK_TOK + 1, :]
    s = (
        jax.lax.dot_general(
            v, W_ref[...], (((1,), (0,)), ((), ())),
            preferred_element_type=jnp.float32,
        )
        + b_ref[...]
    )

    m = jnp.max(s)
    idx = jax.lax.broadcasted_iota(jnp.int32, s.shape, 1)
    a = jnp.min(jnp.where(s == m, idx, _TILE))

    @pl.when(j == 0)
    def _():
        bestv_ref[0] = m
        besti_ref[0] = a

    @pl.when((j > 0) & (m > bestv_ref[0]))
    def _():
        bestv_ref[0] = m
        besti_ref[0] = j * _TILE + a

    @pl.when(j == nj - 1)
    def _():
        p_ref[...] = jnp.full((1, 128), besti_ref[0].astype(jnp.float32))


def _select_body(p_hbm, x_hbm, r_hbm, out_hbm, p_v, x_v, r_v, o_v):
    wid = lax.axis_index("s")
    row = wid // (2048 // _XPW)
    col = (wid % (2048 // _XPW)) * _XPW
    pltpu.sync_copy(p_hbm.at[0], p_v)
    pltpu.sync_copy(x_hbm.at[row, pl.ds(col, _XPW)], x_v)
    pltpu.sync_copy(r_hbm.at[row, pl.ds(col, _XPW)], r_v)
    pb = p_v[pl.ds(0, _L)]

    for k in range(_XPW // _L):
        xk = x_v[pl.ds(k * _L, _L)]
        rk = r_v[pl.ds(k * _L, _L)]
        cond = (xk == jnp.float32(_MASK_TOK)) & (rk < jnp.float32(_ALPHA))
        o_v[pl.ds(k * _L, _L)] = jnp.where(cond, pb, xk)
    pltpu.sync_copy(o_v, out_hbm.at[row, pl.ds(col, _XPW)])


def kernel(X, rand_vals, emb, W, b):
    b2 = b.reshape(1, _VOCAB)
    p_row = pl.pallas_call(
        _matvec_body,
        grid=(_VOCAB // _TILE,),
        in_specs=[
            pl.BlockSpec((8, _D), lambda j: (0, 0)),
            pl.BlockSpec((_D, _TILE), lambda j: (0, j)),
            pl.BlockSpec((1, _TILE), lambda j: (0, j)),
        ],
        out_specs=pl.BlockSpec((1, 128), lambda j: (0, 0)),
        out_shape=jax.ShapeDtypeStruct((1, 128), jnp.float32),
        scratch_shapes=[
            pltpu.SMEM((1,), jnp.float32),
            pltpu.SMEM((1,), jnp.int32),
        ],
    )(emb, W, b2)

    out = pl.kernel(
        _select_body,
        out_type=jax.ShapeDtypeStruct(X.shape, jnp.float32),
        mesh=_mesh,
        scratch_types=[
            pltpu.VMEM((128,), jnp.float32),
            pltpu.VMEM((_XPW,), jnp.float32),
            pltpu.VMEM((_XPW,), jnp.float32),
            pltpu.VMEM((_XPW,), jnp.float32),
        ],
    )(p_row, X, rand_vals)
    return out


# speedup vs baseline: 1.0230x; 1.0230x over previous
import jax
import jax.numpy as jnp
from jax import lax
from jax.experimental import pallas as pl
from jax.experimental.pallas import tpu as pltpu
from jax.experimental.pallas import tpu_sc as plsc

_VOCAB = 8192
_D = 768
_ALPHA = 0.1
_MASK_TOK = 2
_TILE = 2048

_NC, _NS, _L = 1, 8, 16
_NW = _NC * _NS
_BL = 2 * 2048
_XPW = _BL // _NW

_mesh = plsc.VectorSubcoreMesh(core_axis_name="c", subcore_axis_name="s", num_cores=1, num_subcores=8)


def _matvec_body(emb_ref, W_ref, b_ref, p_ref, bestv_ref, besti_ref):
    j = pl.program_id(0)
    nj = pl.num_programs(0)

    v = emb_ref[_MASK_TOK : _MASK_TOK + 1, :]
    s = (
        jax.lax.dot_general(
            v, W_ref[...], (((1,), (0,)), ((), ())),
            preferred_element_type=jnp.float32,
        )
        + b_ref[...]
    )

    m = jnp.max(s)
    idx = jax.lax.broadcasted_iota(jnp.int32, s.shape, 1)
    a = jnp.min(jnp.where(s == m, idx, _TILE))

    @pl.when(j == 0)
    def _():
        bestv_ref[0] = m
        besti_ref[0] = a

    @pl.when((j > 0) & (m > bestv_ref[0]))
    def _():
        bestv_ref[0] = m
        besti_ref[0] = j * _TILE + a

    @pl.when(j == nj - 1)
    def _():
        p_ref[...] = jnp.full((1, 128), besti_ref[0].astype(jnp.float32))


def _select_body(p_hbm, x_hbm, r_hbm, out_hbm, p_v, x_v, r_v, o_v):
    wid = lax.axis_index("s")
    row = wid // (2048 // _XPW)
    col = (wid % (2048 // _XPW)) * _XPW
    pltpu.sync_copy(p_hbm.at[0], p_v)
    pltpu.sync_copy(x_hbm.at[row, pl.ds(col, _XPW)], x_v)
    pltpu.sync_copy(r_hbm.at[row, pl.ds(col, _XPW)], r_v)
    pb = p_v[pl.ds(0, _L)]

    for k in range(_XPW // _L):
        xk = x_v[pl.ds(k * _L, _L)]
        rk = r_v[pl.ds(k * _L, _L)]
        cond = (xk == jnp.float32(_MASK_TOK)) & (rk < jnp.float32(_ALPHA))
        o_v[pl.ds(k * _L, _L)] = jnp.where(cond, pb, xk)
    pltpu.sync_copy(o_v, out_hbm.at[row, pl.ds(col, _XPW)])


def kernel(X, rand_vals, emb, W, b):
    b2 = b.reshape(1, _VOCAB)
    p_row = pl.pallas_call(
        _matvec_body,
        grid=(_VOCAB // _TILE,),
        in_specs=[
            pl.BlockSpec((8, _D), lambda j: (0, 0)),
            pl.BlockSpec((_D, _TILE), lambda j: (0, j)),
            pl.BlockSpec((1, _TILE), lambda j: (0, j)),
        ],
        out_specs=pl.BlockSpec((1, 128), lambda j: (0, 0)),
        out_shape=jax.ShapeDtypeStruct((1, 128), jnp.float32),
        scratch_shapes=[
            pltpu.SMEM((1,), jnp.float32),
            pltpu.SMEM((1,), jnp.int32),
        ],
    )(emb, W, b2)

    out = pl.kernel(
        _select_body,
        out_type=jax.ShapeDtypeStruct(X.shape, jnp.float32),
        mesh=_mesh,
        scratch_types=[
            pltpu.VMEM((128,), jnp.float32),
            pltpu.VMEM((_XPW,), jnp.float32),
            pltpu.VMEM((_XPW,), jnp.float32),
            pltpu.VMEM((_XPW,), jnp.float32),
        ],
    )(p_row, X, rand_vals)
    return out


# speedup vs baseline: 1.0440x; 1.0206x over previous
import jax
import jax.numpy as jnp
from jax import lax
from jax.experimental import pallas as pl
from jax.experimental.pallas import tpu as pltpu
from jax.experimental.pallas import tpu_sc as plsc

_VOCAB = 8192
_D = 768
_ALPHA = 0.1
_MASK_TOK = 2
_TILE = 2048

_NC, _NS, _L = 1, 8, 16
_NW = _NC * _NS
_BL = 2 * 2048
_XPW = _BL // _NW

_mesh = plsc.VectorSubcoreMesh(core_axis_name="c", subcore_axis_name="s", num_cores=1, num_subcores=8)


def _matvec_body(emb_ref, W_ref, b_ref, p_ref, bestv_ref, besti_ref):
    j = pl.program_id(0)
    nj = pl.num_programs(0)

    v = emb_ref[_MASK_TOK : _MASK_TOK + 1, :]
    s = (
        jax.lax.dot_general(
            v, W_ref[...], (((1,), (0,)), ((), ())),
            preferred_element_type=jnp.float32,
        )
        + b_ref[...]
    )

    m = jnp.max(s)
    idx = jax.lax.broadcasted_iota(jnp.int32, s.shape, 1)
    a = jnp.min(jnp.where(s == m, idx, _TILE))

    @pl.when(j == 0)
    def _():
        bestv_ref[0] = m
        besti_ref[0] = a

    @pl.when((j > 0) & (m > bestv_ref[0]))
    def _():
        bestv_ref[0] = m
        besti_ref[0] = j * _TILE + a

    @pl.when(j == nj - 1)
    def _():
        p_ref[...] = jnp.full((1, 128), besti_ref[0].astype(jnp.float32))


def _select_body(p_hbm, x_hbm, r_hbm, out_hbm, p_v, x_v, r_v, o_v,
                 semp, semx, semr):
    wid = lax.axis_index("s")
    row = wid // (2048 // _XPW)
    col = (wid % (2048 // _XPW)) * _XPW
    cp_p = pltpu.async_copy(p_hbm.at[0], p_v, semp)
    cp_x = pltpu.async_copy(x_hbm.at[row, pl.ds(col, _XPW)], x_v, semx)
    cp_r = pltpu.async_copy(r_hbm.at[row, pl.ds(col, _XPW)], r_v, semr)
    cp_p.wait()
    cp_x.wait()
    cp_r.wait()
    pb = p_v[pl.ds(0, _L)]

    for k in range(_XPW // _L):
        xk = x_v[pl.ds(k * _L, _L)]
        rk = r_v[pl.ds(k * _L, _L)]
        cond = (xk == jnp.float32(_MASK_TOK)) & (rk < jnp.float32(_ALPHA))
        o_v[pl.ds(k * _L, _L)] = jnp.where(cond, pb, xk)
    pltpu.sync_copy(o_v, out_hbm.at[row, pl.ds(col, _XPW)])


def kernel(X, rand_vals, emb, W, b):
    b2 = b.reshape(1, _VOCAB)
    p_row = pl.pallas_call(
        _matvec_body,
        grid=(_VOCAB // _TILE,),
        in_specs=[
            pl.BlockSpec((8, _D), lambda j: (0, 0)),
            pl.BlockSpec((_D, _TILE), lambda j: (0, j)),
            pl.BlockSpec((1, _TILE), lambda j: (0, j)),
        ],
        out_specs=pl.BlockSpec((1, 128), lambda j: (0, 0)),
        out_shape=jax.ShapeDtypeStruct((1, 128), jnp.float32),
        scratch_shapes=[
            pltpu.SMEM((1,), jnp.float32),
            pltpu.SMEM((1,), jnp.int32),
        ],
    )(emb, W, b2)

    out = pl.kernel(
        _select_body,
        out_type=jax.ShapeDtypeStruct(X.shape, jnp.float32),
        mesh=_mesh,
        scratch_types=[
            pltpu.VMEM((128,), jnp.float32),
            pltpu.VMEM((_XPW,), jnp.float32),
            pltpu.VMEM((_XPW,), jnp.float32),
            pltpu.VMEM((_XPW,), jnp.float32),
            pltpu.SemaphoreType.DMA,
            pltpu.SemaphoreType.DMA,
            pltpu.SemaphoreType.DMA,
        ],
    )(p_row, X, rand_vals)
    return out
